# initial kernel scaffold (unmeasured)
import jax
import jax.numpy as jnp
from jax import lax
from jax.experimental import pallas as pl
from jax.experimental.pallas import tpu as pltpu

NY = 4
NZ = 4
NRING = NY * NZ
NX = 2


def kernel(x, dy):
    m, d = x.shape
    _, f = dy.shape
    fb = f // NRING
    dh = d // NX

    def body(x_ref, dy_ref, out_ref,
             dys_ref, csend_ref, xrecv_ref,
             dy_copy_sem, xsend_sem, xrecv_sem, ring_send_sems, ring_recv_sems):
        gx = lax.axis_index("x")
        gy = lax.axis_index("y")
        gz = lax.axis_index("z")

        r = gy * NZ + jnp.where(gy % 2 == 0, gz, NZ - 1 - gz)

        def ring_coords(pos):
            ry = pos // NZ
            raw = pos % NZ
            rz = jnp.where(ry % 2 == 0, raw, NZ - 1 - raw)
            return ry, rz

        right_y, right_z = ring_coords((r + 1) % NRING)
        left_y, left_z = ring_coords((r - 1) % NRING)

        dy_cp = pltpu.make_async_copy(
            dy_ref.at[:, pl.ds(r * fb, fb)], dys_ref, dy_copy_sem
        )
        dy_cp.start()
        dy_cp.wait()

        dims = (((0,), (0,)), ((), ()))

        peer = 1 - gx
        csend_ref[:, :] = lax.dot_general(
            x_ref[:, pl.ds(peer * dh, dh)], dys_ref[:, :], dims,
            preferred_element_type=jnp.float32,
        )
        xchg = pltpu.make_async_remote_copy(
            src_ref=csend_ref,
            dst_ref=xrecv_ref,
            send_sem=xsend_sem,
            recv_sem=xrecv_sem,
            device_id=(peer, gy, gz),
            device_id_type=pltpu.DeviceIdType.MESH,
        )
        xchg.start()

        c_mine = lax.dot_general(
            x_ref[:, pl.ds(gx * dh, dh)], dys_ref[:, :], dims,
            preferred_element_type=jnp.float32,
        )
        xchg.wait()

        out_ref[:, pl.ds(r * fb, fb)] = c_mine + xrecv_ref[:, :]

        for h in range(NRING - 1):
            p_send = (r - h) % NRING
            p_recv = (r - h - 1) % NRING
            send = pltpu.make_async_remote_copy(
                src_ref=out_ref.at[:, pl.ds(p_send * fb, fb)],
                dst_ref=out_ref.at[:, pl.ds(p_send * fb, fb)],
                send_sem=ring_send_sems.at[h],
                recv_sem=ring_recv_sems.at[h],
                device_id=(gx, right_y, right_z),
                device_id_type=pltpu.DeviceIdType.MESH,
            )
            send.start()
            recv = pltpu.make_async_remote_copy(
                src_ref=out_ref.at[:, pl.ds(p_recv * fb, fb)],
                dst_ref=out_ref.at[:, pl.ds(p_recv * fb, fb)],
                send_sem=ring_send_sems.at[h],
                recv_sem=ring_recv_sems.at[h],
                device_id=(gx, left_y, left_z),
                device_id_type=pltpu.DeviceIdType.MESH,
            )
            recv.wait_recv()
            send.wait_send()

    return pl.pallas_call(
        body,
        out_shape=jax.ShapeDtypeStruct((dh, f), jnp.float32),
        in_specs=[
            pl.BlockSpec(memory_space=pltpu.VMEM),
            pl.BlockSpec(memory_space=pltpu.ANY),
        ],
        out_specs=pl.BlockSpec(memory_space=pltpu.VMEM),
        scratch_shapes=[
            pltpu.VMEM((m, fb), jnp.float32),
            pltpu.VMEM((dh, fb), jnp.float32),
            pltpu.VMEM((dh, fb), jnp.float32),
            pltpu.SemaphoreType.DMA,
            pltpu.SemaphoreType.DMA,
            pltpu.SemaphoreType.DMA,
            pltpu.SemaphoreType.DMA((NRING - 1,)),
            pltpu.SemaphoreType.DMA((NRING - 1,)),
        ],
    )(x, dy)


# baseline (device time: 442778 ns/iter reference)
import jax
import jax.numpy as jnp
from jax import lax
from jax.experimental import pallas as pl
from jax.experimental.pallas import tpu as pltpu

NY = 4
NZ = 4
NRING = NY * NZ
NX = 2


def kernel(x, dy):
    m, d = x.shape
    _, f = dy.shape
    fb = f // NRING
    dh = d // NX

    def body(x_ref, dy_ref, out_ref,
             xh_ref, dys_ref, csend_ref, xrecv_ref,
             dy_copy_sem, xh_copy_sem, xsend_sem, xrecv_sem,
             ring_send_sems, ring_recv_sems):
        gx = lax.axis_index("x")
        gy = lax.axis_index("y")
        gz = lax.axis_index("z")
        peer = 1 - gx

        r = gy * NZ + jnp.where(gy % 2 == 0, gz, NZ - 1 - gz)

        def ring_coords(pos):
            ry = pos // NZ
            raw = pos % NZ
            rz = jnp.where(ry % 2 == 0, raw, NZ - 1 - raw)
            return ry, rz

        right_y, right_z = ring_coords((r + 1) % NRING)
        left_y, left_z = ring_coords((r - 1) % NRING)

        dy_cp = pltpu.make_async_copy(
            dy_ref.at[:, pl.ds(r * fb, fb)], dys_ref, dy_copy_sem
        )
        dy_cp.start()
        xh_cp = pltpu.make_async_copy(
            x_ref.at[:, pl.ds(peer * dh, dh)], xh_ref, xh_copy_sem
        )
        xh_cp.start()
        dy_cp.wait()
        xh_cp.wait()

        dims = (((0,), (0,)), ((), ()))

        csend_ref[:, :] = lax.dot_general(
            xh_ref[:, :], dys_ref[:, :], dims,
            preferred_element_type=jnp.float32,
        )
        xchg = pltpu.make_async_remote_copy(
            src_ref=csend_ref,
            dst_ref=xrecv_ref,
            send_sem=xsend_sem,
            recv_sem=xrecv_sem,
            device_id=(peer, gy, gz),
            device_id_type=pl.DeviceIdType.MESH,
        )
        xchg.start()

        xh2_cp = pltpu.make_async_copy(
            x_ref.at[:, pl.ds(gx * dh, dh)], xh_ref, xh_copy_sem
        )
        xh2_cp.start()
        xh2_cp.wait()
        c_mine = lax.dot_general(
            xh_ref[:, :], dys_ref[:, :], dims,
            preferred_element_type=jnp.float32,
        )
        xchg.wait()

        out_ref[:, pl.ds(r * fb, fb)] = c_mine + xrecv_ref[:, :]

        for h in range(NRING - 1):
            p_send = (r - h) % NRING
            p_recv = (r - h - 1) % NRING
            send = pltpu.make_async_remote_copy(
                src_ref=out_ref.at[:, pl.ds(p_send * fb, fb)],
                dst_ref=out_ref.at[:, pl.ds(p_send * fb, fb)],
                send_sem=ring_send_sems.at[h],
                recv_sem=ring_recv_sems.at[h],
                device_id=(gx, right_y, right_z),
                device_id_type=pl.DeviceIdType.MESH,
            )
            send.start()
            recv = pltpu.make_async_remote_copy(
                src_ref=out_ref.at[:, pl.ds(p_recv * fb, fb)],
                dst_ref=out_ref.at[:, pl.ds(p_recv * fb, fb)],
                send_sem=ring_send_sems.at[h],
                recv_sem=ring_recv_sems.at[h],
                device_id=(gx, left_y, left_z),
                device_id_type=pl.DeviceIdType.MESH,
            )
            recv.wait_recv()
            send.wait_send()

    return pl.pallas_call(
        body,
        out_shape=jax.ShapeDtypeStruct((dh, f), jnp.float32),
        in_specs=[
            pl.BlockSpec(memory_space=pl.ANY),
            pl.BlockSpec(memory_space=pl.ANY),
        ],
        out_specs=pl.BlockSpec(memory_space=pltpu.MemorySpace.VMEM),
        scratch_shapes=[
            pltpu.VMEM((m, dh), jnp.float32),
            pltpu.VMEM((m, fb), jnp.float32),
            pltpu.VMEM((dh, fb), jnp.float32),
            pltpu.VMEM((dh, fb), jnp.float32),
            pltpu.SemaphoreType.DMA,
            pltpu.SemaphoreType.DMA,
            pltpu.SemaphoreType.DMA,
            pltpu.SemaphoreType.DMA,
            pltpu.SemaphoreType.DMA((NRING - 1,)),
            pltpu.SemaphoreType.DMA((NRING - 1,)),
        ],
        compiler_params=pltpu.CompilerParams(
            vmem_limit_bytes=60 * 1024 * 1024,
        ),
    )(x, dy)


# device time: 260303 ns/iter; 1.7010x vs baseline; 1.7010x over previous
import jax
import jax.numpy as jnp
from jax import lax
from jax.experimental import pallas as pl
from jax.experimental.pallas import tpu as pltpu

NY = 4
NZ = 4
NX = 2
NLINE = 4
NSTEP = NLINE - 1


def kernel(x, dy):
    m, d = x.shape
    _, f = dy.shape
    fb = f // (NY * NZ)
    half = fb // 2
    dh = d // NX

    def body(x_ref, dy_ref, out_ref,
             xh_ref, dys_ref, csend_ref, xrecv_ref,
             dy_copy_sem, xh_copy_sem, xsend_sem, xrecv_sem,
             ag_send_sems, ag_recv_sems):
        gx = lax.axis_index("x")
        gy = lax.axis_index("y")
        gz = lax.axis_index("z")
        peer = 1 - gx
        b = gy * NZ + gz

        dy_cp = pltpu.make_async_copy(
            dy_ref.at[:, pl.ds(b * fb, fb)], dys_ref, dy_copy_sem
        )
        dy_cp.start()
        xh_cp = pltpu.make_async_copy(
            x_ref.at[:, pl.ds(peer * dh, dh)], xh_ref, xh_copy_sem
        )
        xh_cp.start()
        dy_cp.wait()
        xh_cp.wait()

        dims = (((0,), (0,)), ((), ()))
        csend_ref[:, :] = lax.dot_general(
            xh_ref[:, :], dys_ref[:, :], dims,
            preferred_element_type=jnp.float32,
        )
        xchg = pltpu.make_async_remote_copy(
            src_ref=csend_ref,
            dst_ref=xrecv_ref,
            send_sem=xsend_sem,
            recv_sem=xrecv_sem,
            device_id=(peer, gy, gz),
            device_id_type=pl.DeviceIdType.MESH,
        )
        xchg.start()

        xh2_cp = pltpu.make_async_copy(
            x_ref.at[:, pl.ds(gx * dh, dh)], xh_ref, xh_copy_sem
        )
        xh2_cp.start()
        xh2_cp.wait()
        c_mine = lax.dot_general(
            xh_ref[:, :], dys_ref[:, :], dims,
            preferred_element_type=jnp.float32,
        )
        xchg.wait()
        out_ref[:, pl.ds(b * fb, fb)] = c_mine + xrecv_ref[:, :]

        def slices_a1(k):
            return [((gy * NZ + k) * fb, half)]

        def slices_b1(k):
            return [((k * NZ + gz) * fb + half, half)]

        def slices_a2(k):
            return [((k * NZ + t) * fb, half) for t in range(NZ)]

        def slices_b2(k):
            return [((t * NZ + k) * fb + half, half) for t in range(NY)]

        def nbr_y(delta):
            return (gx, gy + delta, gz)

        def nbr_z(delta):
            return (gx, gy, gz + delta)

        configs = {
            "a1": (gz, slices_a1, nbr_z, 0),
            "b1": (gy, slices_b1, nbr_y, 2 * NSTEP),
            "a2": (gy, slices_a2, nbr_y, 4 * NSTEP),
            "b2": (gz, slices_b2, nbr_z, 6 * NSTEP),
        }

        def descs(cfg, k, delta, sem_idx):
            _, slices_fn, nbr_fn, _ = configs[cfg]
            out = []
            for off, w in slices_fn(k):
                out.append(pltpu.make_async_remote_copy(
                    src_ref=out_ref.at[:, pl.ds(off, w)],
                    dst_ref=out_ref.at[:, pl.ds(off, w)],
                    send_sem=ag_send_sems.at[sem_idx],
                    recv_sem=ag_recv_sems.at[sem_idx],
                    device_id=nbr_fn(delta),
                    device_id_type=pl.DeviceIdType.MESH,
                ))
            return out

        def step_conds(cfg, s):
            pos = configs[cfg][0]
            base = configs[cfg][3]
            return [
                dict(dir=0, sem=base + s,
                     send_cond=(pos < NLINE - 1) & (pos - s >= 0),
                     send_k=pos - s,
                     recv_cond=(pos - 1 - s >= 0),
                     recv_k=pos - 1 - s,
                     delta=1),
                dict(dir=1, sem=base + NSTEP + s,
                     send_cond=(pos > 0) & (pos + s <= NLINE - 1),
                     send_k=pos + s,
                     recv_cond=(pos + 1 + s <= NLINE - 1),
                     recv_k=pos + 1 + s,
                     delta=-1),
            ]

        def run_phase(cfgs):
            for s in range(NSTEP):
                plan = [(cfg, io) for cfg in cfgs for io in step_conds(cfg, s)]
                for cfg, io in plan:
                    @pl.when(io["send_cond"])
                    def _(cfg=cfg, io=io):
                        for rd in descs(cfg, io["send_k"], io["delta"],
                                        io["sem"]):
                            rd.start()
                for cfg, io in plan:
                    @pl.when(io["recv_cond"])
                    def _(cfg=cfg, io=io):
                        for rd in descs(cfg, io["recv_k"], -io["delta"],
                                        io["sem"]):
                            rd.wait_recv()
                for cfg, io in plan:
                    @pl.when(io["send_cond"])
                    def _(cfg=cfg, io=io):
                        for rd in descs(cfg, io["send_k"], io["delta"],
                                        io["sem"]):
                            rd.wait_send()

        run_phase(["a1", "b1"])
        run_phase(["a2", "b2"])

    return pl.pallas_call(
        body,
        out_shape=jax.ShapeDtypeStruct((dh, f), jnp.float32),
        in_specs=[
            pl.BlockSpec(memory_space=pl.ANY),
            pl.BlockSpec(memory_space=pl.ANY),
        ],
        out_specs=pl.BlockSpec(memory_space=pltpu.MemorySpace.VMEM),
        scratch_shapes=[
            pltpu.VMEM((m, dh), jnp.float32),
            pltpu.VMEM((m, fb), jnp.float32),
            pltpu.VMEM((dh, fb), jnp.float32),
            pltpu.VMEM((dh, fb), jnp.float32),
            pltpu.SemaphoreType.DMA,
            pltpu.SemaphoreType.DMA,
            pltpu.SemaphoreType.DMA,
            pltpu.SemaphoreType.DMA,
            pltpu.SemaphoreType.DMA((8 * NSTEP,)),
            pltpu.SemaphoreType.DMA((8 * NSTEP,)),
        ],
        compiler_params=pltpu.CompilerParams(
            vmem_limit_bytes=60 * 1024 * 1024,
        ),
    )(x, dy)


# device time: 244577 ns/iter; 1.8104x vs baseline; 1.0643x over previous
import jax
import jax.numpy as jnp
from jax import lax
from jax.experimental import pallas as pl
from jax.experimental.pallas import tpu as pltpu

NY = 4
NZ = 4
NX = 2
NLINE = 4
NSTEP = NLINE - 1
NQ = 4


def kernel(x, dy):
    m, d = x.shape
    _, f = dy.shape
    fb = f // (NY * NZ)
    qw = fb // NQ
    dh = d // NX

    def body(x_ref, dy_ref, out_ref,
             xh_ref, dys_ref, csend_ref, xrecv_ref,
             dy_copy_sem, xh_copy_sem, xsend_sems, xrecv_sems,
             ag_send_sems, ag_recv_sems):
        gx = lax.axis_index("x")
        gy = lax.axis_index("y")
        gz = lax.axis_index("z")
        peer = 1 - gx
        b = gy * NZ + gz

        def nbr_y(delta):
            return (gx, gy + delta, gz)

        def nbr_z(delta):
            return (gx, gy, gz + delta)

        def cfg(q, p):
            zfirst = (q % 2 == 0)
            on_z = (p == 0) == zfirst
            sem_base = (q * 2 + p) * 2 * NSTEP
            if p == 0:
                if zfirst:
                    sl = lambda k: [((gy * NZ + k) * fb + q * qw, qw)]
                else:
                    sl = lambda k: [((k * NZ + gz) * fb + q * qw, qw)]
            else:
                if zfirst:
                    sl = lambda k: [((k * NZ + t) * fb + q * qw, qw)
                                    for t in range(NZ)]
                else:
                    sl = lambda k: [((t * NZ + k) * fb + q * qw, qw)
                                    for t in range(NY)]
            pos = gz if on_z else gy
            nbr = nbr_z if on_z else nbr_y
            return pos, sl, nbr, sem_base

        def descs(q, p, k, delta, sem_idx):
            _, sl, nbr, _ = cfg(q, p)
            return [
                pltpu.make_async_remote_copy(
                    src_ref=out_ref.at[:, pl.ds(off, w)],
                    dst_ref=out_ref.at[:, pl.ds(off, w)],
                    send_sem=ag_send_sems.at[sem_idx],
                    recv_sem=ag_recv_sems.at[sem_idx],
                    device_id=nbr(delta),
                    device_id_type=pl.DeviceIdType.MESH,
                )
                for off, w in sl(k)
            ]

        def step_ios(q, p, s):
            pos, _, _, base = cfg(q, p)
            return [
                dict(sem=base + s,
                     send_cond=(pos < NLINE - 1) & (pos - s >= 0),
                     send_k=pos - s,
                     recv_cond=(pos - 1 - s >= 0),
                     recv_k=pos - 1 - s,
                     delta=1),
                dict(sem=base + NSTEP + s,
                     send_cond=(pos > 0) & (pos + s <= NLINE - 1),
                     send_k=pos + s,
                     recv_cond=(pos + 1 + s <= NLINE - 1),
                     recv_k=pos + 1 + s,
                     delta=-1),
            ]

        def ag_send(q, p, s):
            for io in step_ios(q, p, s):
                @pl.when(io["send_cond"])
                def _(io=io):
                    for rd in descs(q, p, io["send_k"], io["delta"],
                                    io["sem"]):
                        rd.start()

        def ag_recv_wait(q, p, s):
            for io in step_ios(q, p, s):
                @pl.when(io["recv_cond"])
                def _(io=io):
                    for rd in descs(q, p, io["recv_k"], -io["delta"],
                                    io["sem"]):
                        rd.wait_recv()

        def ag_send_wait(q, p, s):
            for io in step_ios(q, p, s):
                @pl.when(io["send_cond"])
                def _(io=io):
                    for rd in descs(q, p, io["send_k"], io["delta"],
                                    io["sem"]):
                        rd.wait_send()

        dy_cp = pltpu.make_async_copy(
            dy_ref.at[:, pl.ds(b * fb, fb)], dys_ref, dy_copy_sem
        )
        dy_cp.start()
        xh_cp = pltpu.make_async_copy(
            x_ref.at[:, pl.ds(peer * dh, dh)], xh_ref, xh_copy_sem
        )
        xh_cp.start()
        dy_cp.wait()
        xh_cp.wait()

        dims = (((0,), (0,)), ((), ()))

        def xchg_desc(q):
            return pltpu.make_async_remote_copy(
                src_ref=csend_ref.at[:, pl.ds(q * qw, qw)],
                dst_ref=xrecv_ref.at[:, pl.ds(q * qw, qw)],
                send_sem=xsend_sems.at[q],
                recv_sem=xrecv_sems.at[q],
                device_id=(peer, gy, gz),
                device_id_type=pl.DeviceIdType.MESH,
            )

        for q in range(NQ):
            csend_ref[:, q * qw:(q + 1) * qw] = lax.dot_general(
                xh_ref[:, :], dys_ref[:, q * qw:(q + 1) * qw], dims,
                preferred_element_type=jnp.float32,
            )
            xchg_desc(q).start()

        xh2_cp = pltpu.make_async_copy(
            x_ref.at[:, pl.ds(gx * dh, dh)], xh_ref, xh_copy_sem
        )
        xh2_cp.start()
        xh2_cp.wait()

        for q in range(NQ):
            c_mine = lax.dot_general(
                xh_ref[:, :], dys_ref[:, q * qw:(q + 1) * qw], dims,
                preferred_element_type=jnp.float32,
            )
            xchg_desc(q).wait()
            out_ref[:, pl.ds(b * fb + q * qw, qw)] = \
                c_mine + xrecv_ref[:, q * qw:(q + 1) * qw]
            ag_send(q, 0, 0)

        for s in range(NSTEP):
            for q in range(NQ):
                ag_recv_wait(q, 0, s)
                ag_send_wait(q, 0, s)
                if s + 1 < NSTEP:
                    ag_send(q, 0, s + 1)
                else:
                    ag_send(q, 1, 0)
        for s in range(NSTEP):
            for q in range(NQ):
                ag_recv_wait(q, 1, s)
                ag_send_wait(q, 1, s)
                if s + 1 < NSTEP:
                    ag_send(q, 1, s + 1)

    return pl.pallas_call(
        body,
        out_shape=jax.ShapeDtypeStruct((dh, f), jnp.float32),
        in_specs=[
            pl.BlockSpec(memory_space=pl.ANY),
            pl.BlockSpec(memory_space=pl.ANY),
        ],
        out_specs=pl.BlockSpec(memory_space=pltpu.MemorySpace.VMEM),
        scratch_shapes=[
            pltpu.VMEM((m, dh), jnp.float32),
            pltpu.VMEM((m, fb), jnp.float32),
            pltpu.VMEM((dh, fb), jnp.float32),
            pltpu.VMEM((dh, fb), jnp.float32),
            pltpu.SemaphoreType.DMA,
            pltpu.SemaphoreType.DMA,
            pltpu.SemaphoreType.DMA((NQ,)),
            pltpu.SemaphoreType.DMA((NQ,)),
            pltpu.SemaphoreType.DMA((NQ * 2 * 2 * NSTEP,)),
            pltpu.SemaphoreType.DMA((NQ * 2 * 2 * NSTEP,)),
        ],
        compiler_params=pltpu.CompilerParams(
            vmem_limit_bytes=60 * 1024 * 1024,
        ),
    )(x, dy)
